# baseline (speedup 1.0000x reference)
import functools

import jax
import jax.numpy as jnp
from jax.experimental import pallas as pl
from jax.experimental.pallas import tpu as pltpu
from jax.experimental.pallas import tpu_sc as plsc

_NEG = -2.0
_BIG = 3.0e38
_KAUG = 136


def _augment_kernel(x_ref, y_ref, xa_ref, ya_ref, *, bp, d):
    x = x_ref[...]
    x2 = jnp.sum(x * x, axis=1, keepdims=True)
    ones = jnp.ones((bp, 1), jnp.float32)
    zeros = jnp.zeros((bp, _KAUG - d - 2), jnp.float32)
    xa_ref[...] = jnp.concatenate([_NEG * x, x2, ones, zeros], axis=1)
    y = y_ref[...]
    y2 = jnp.sum(y * y, axis=1, keepdims=True)
    ya_ref[...] = jnp.concatenate([y, ones, y2, zeros], axis=1)


def _augment(x, y, *, bp=512):
    n, d = x.shape
    out = pl.pallas_call(
        functools.partial(_augment_kernel, bp=bp, d=d),
        grid=(n // bp,),
        in_specs=[
            pl.BlockSpec((bp, d), lambda i: (i, 0)),
            pl.BlockSpec((bp, d), lambda i: (i, 0)),
        ],
        out_specs=[
            pl.BlockSpec((bp, _KAUG), lambda i: (i, 0)),
            pl.BlockSpec((bp, _KAUG), lambda i: (i, 0)),
        ],
        out_shape=[
            jax.ShapeDtypeStruct((n, _KAUG), jnp.float32),
            jax.ShapeDtypeStruct((n, _KAUG), jnp.float32),
        ],
    )(x, y)
    return out


def _main_kernel(xa_ref, ya_ref, p_ref, l1_ref, min1_ref, arg_ref,
                 cmin_ref, rmin_ref, ridx_ref, l1s_ref, *, bi, bj):
    i = pl.program_id(0)
    j = pl.program_id(1)
    ni = pl.num_programs(0)
    nj = pl.num_programs(1)
    ns = bj // 128

    @pl.when(jnp.logical_and(i == 0, j == 0))
    def _():
        cmin_ref[...] = jnp.full_like(cmin_ref, _BIG)
        l1s_ref[0, 0] = 0.0

    xa = xa_ref[...]

    @pl.when(j == 0)
    def _():
        rmin_ref[...] = jnp.full_like(rmin_ref, _BIG)
        ridx_ref[...] = jnp.zeros_like(ridx_ref)

    rm = rmin_ref[...]
    ri = ridx_ref[...]
    for s in range(ns):
        ya = ya_ref[pl.ds(j * bj + s * 128, 128), :]
        d_s = jax.lax.dot_general(
            xa, ya, (((1,), (1,)), ((), ())),
            preferred_element_type=jnp.float32)
        cpart = jnp.min(d_s.reshape(bi // 8, 8, 128), axis=0)
        cs = cmin_ref[:, pl.ds(j * bj + s * 128, 128)]
        cmin_ref[:, pl.ds(j * bj + s * 128, 128)] = jnp.minimum(cs, cpart)
        better = d_s < rm
        rm = jnp.where(better, d_s, rm)
        ri = jnp.where(better, jnp.int32(j * ns + s), ri)
    rmin_ref[...] = rm
    ridx_ref[...] = ri

    @pl.when(j == nj - 1)
    def _():
        m = jnp.min(rm, axis=1)
        lane = jax.lax.broadcasted_iota(jnp.int32, (bi, 128), 1)
        gidx = ri * 128 + lane
        am = jnp.min(jnp.where(rm == m[:, None], gidx, jnp.int32(2 ** 30)),
                     axis=1)
        min1_ref[0, pl.ds(i * bi, bi)] = jnp.maximum(m, 0.0)
        arg_ref[0, pl.ds(i * bi, bi)] = am

    @pl.when(i == ni - 1)
    def _():
        cm8 = cmin_ref[:, pl.ds(j * bj, bj)]
        cmin1 = jnp.maximum(jnp.min(cm8, axis=0), 0.0)
        pj = p_ref[0, pl.ds(j * bj, bj)]
        l1s_ref[0, 0] += jnp.sum(pj * cmin1)

    @pl.when(jnp.logical_and(i == ni - 1, j == nj - 1))
    def _():
        l1_ref[...] = jnp.reshape(l1s_ref[0, 0], (1, 1))


def _chamfer_main(xa, ya, probs, *, bi, bj, interpret=False):
    n = xa.shape[0]
    m = ya.shape[0]
    grid = (n // bi, m // bj)
    l1, min1, arg1 = pl.pallas_call(
        functools.partial(_main_kernel, bi=bi, bj=bj),
        grid=grid,
        in_specs=[
            pl.BlockSpec((bi, _KAUG), lambda i, j: (i, 0)),
            pl.BlockSpec((m, _KAUG), lambda i, j: (0, 0)),
            pl.BlockSpec((1, m), lambda i, j: (0, 0)),
        ],
        out_specs=[
            pl.BlockSpec((1, 1), lambda i, j: (0, 0)),
            pl.BlockSpec((1, n), lambda i, j: (0, 0)),
            pl.BlockSpec((1, n), lambda i, j: (0, 0)),
        ],
        out_shape=[
            jax.ShapeDtypeStruct((1, 1), jnp.float32),
            jax.ShapeDtypeStruct((1, n), jnp.float32),
            jax.ShapeDtypeStruct((1, n), jnp.int32),
        ],
        scratch_shapes=[
            pltpu.VMEM((8, m), jnp.float32),
            pltpu.VMEM((bi, 128), jnp.float32),
            pltpu.VMEM((bi, 128), jnp.int32),
            pltpu.SMEM((1, 1), jnp.float32),
        ],
        interpret=interpret,
    )(xa, ya, probs.reshape(1, m))
    return l1, min1, arg1


def _l2_sc(probs, min1, arg1):
    n = min1.shape[0]
    m = probs.shape[0]
    info = plsc.get_sparse_core_info()
    nc, ns = info.num_cores, info.num_subcores
    nw = nc * ns
    per = n // nw
    mesh = plsc.VectorSubcoreMesh(core_axis_name="c", subcore_axis_name="s")

    nchunk = per // 128

    @functools.partial(
        pl.kernel, mesh=mesh,
        out_type=jax.ShapeDtypeStruct((nw, 16), jnp.float32),
        scratch_types=[
            pltpu.VMEM((nchunk, 128), jnp.int32),
            pltpu.VMEM((per,), jnp.float32),
            pltpu.VMEM((per,), jnp.float32),
            pltpu.VMEM((16,), jnp.float32),
        ],
    )
    def k(probs_hbm, min1_hbm, arg_hbm, out_hbm, idx_v, g_v, m_v, acc_v):
        wid = jax.lax.axis_index("s") * nc + jax.lax.axis_index("c")
        base = wid * per
        pltpu.sync_copy(arg_hbm.at[pl.ds(wid * nchunk, nchunk)], idx_v)
        pltpu.sync_copy(min1_hbm.at[pl.ds(base, per)], m_v)
        for c in range(nchunk):
            pltpu.sync_copy(probs_hbm.at[idx_v.at[c]],
                            g_v.at[pl.ds(c * 128, 128)])

        def body(t, acc):
            g16 = g_v[pl.ds(t * 16, 16)]
            m16 = m_v[pl.ds(t * 16, 16)]
            return acc + g16 * m16

        acc_v[...] = jax.lax.fori_loop(0, per // 16, body,
                                       jnp.zeros((16,), jnp.float32))
        pltpu.sync_copy(acc_v, out_hbm.at[wid])

    return k(probs, min1, arg1.reshape(nw * nchunk, 128))


@jax.jit
def kernel(input_vertex_set, sampled_points, probs):
    n = input_vertex_set.shape[0]
    m = sampled_points.shape[0]
    xa, ya = _augment(input_vertex_set, sampled_points)
    bi = 512 if n % 512 == 0 else 256
    bj = m
    l1, min1, arg1 = _chamfer_main(xa, ya, probs, bi=bi, bj=bj)
    l2p = _l2_sc(probs, min1[0], arg1[0])
    return l1[0, 0] + jnp.sum(l2p)


# speedup vs baseline: 1.0467x; 1.0467x over previous
import functools

import jax
import jax.numpy as jnp
from jax.experimental import pallas as pl
from jax.experimental.pallas import tpu as pltpu
from jax.experimental.pallas import tpu_sc as plsc

_NEG = -2.0
_BIG = 3.0e38
_KAUG = 136


def _augment_kernel(x_ref, y_ref, xa_ref, ya_ref, *, bp, d):
    x = x_ref[...]
    x2 = jnp.sum(x * x, axis=1, keepdims=True)
    ones = jnp.ones((bp, 1), jnp.float32)
    zeros = jnp.zeros((bp, _KAUG - d - 2), jnp.float32)
    xa_ref[...] = jnp.concatenate([_NEG * x, x2, ones, zeros], axis=1)
    y = y_ref[...]
    y2 = jnp.sum(y * y, axis=1, keepdims=True)
    ya_ref[...] = jnp.concatenate([y, ones, y2, zeros], axis=1)


def _augment(x, y, *, bp=512):
    n, d = x.shape
    out = pl.pallas_call(
        functools.partial(_augment_kernel, bp=bp, d=d),
        grid=(n // bp,),
        in_specs=[
            pl.BlockSpec((bp, d), lambda i: (i, 0)),
            pl.BlockSpec((bp, d), lambda i: (i, 0)),
        ],
        out_specs=[
            pl.BlockSpec((bp, _KAUG), lambda i: (i, 0)),
            pl.BlockSpec((bp, _KAUG), lambda i: (i, 0)),
        ],
        out_shape=[
            jax.ShapeDtypeStruct((n, _KAUG), jnp.float32),
            jax.ShapeDtypeStruct((n, _KAUG), jnp.float32),
        ],
    )(x, y)
    return out


def _main_kernel(xa_ref, ya_ref, p_ref, l1_ref, min1_ref, arg_ref,
                 cmin_ref, rmin_ref, ridx_ref, l1s_ref, *, bi, bj):
    i = pl.program_id(0)
    j = pl.program_id(1)
    ni = pl.num_programs(0)
    nj = pl.num_programs(1)
    ns = bj // 128

    @pl.when(jnp.logical_and(i == 0, j == 0))
    def _():
        cmin_ref[...] = jnp.full_like(cmin_ref, _BIG)
        l1s_ref[0, 0] = 0.0

    xa = xa_ref[...]

    @pl.when(j == 0)
    def _():
        rmin_ref[...] = jnp.full_like(rmin_ref, _BIG)
        ridx_ref[...] = jnp.zeros_like(ridx_ref)

    rm = rmin_ref[...]
    ri = ridx_ref[...]
    for s in range(ns):
        ya = ya_ref[pl.ds(j * bj + s * 128, 128), :]
        d_s = jax.lax.dot_general(
            xa, ya, (((1,), (1,)), ((), ())),
            preferred_element_type=jnp.float32)
        cpart = jnp.min(d_s.reshape(bi // 8, 8, 128), axis=0)
        cs = cmin_ref[:, pl.ds(j * bj + s * 128, 128)]
        cmin_ref[:, pl.ds(j * bj + s * 128, 128)] = jnp.minimum(cs, cpart)
        better = d_s < rm
        rm = jnp.where(better, d_s, rm)
        ri = jnp.where(better, jnp.int32(j * ns + s), ri)
    rmin_ref[...] = rm
    ridx_ref[...] = ri

    @pl.when(j == nj - 1)
    def _():
        m = jnp.min(rm, axis=1)
        lane = jax.lax.broadcasted_iota(jnp.int32, (bi, 128), 1)
        gidx = ri * 128 + lane
        am = jnp.min(jnp.where(rm == m[:, None], gidx, jnp.int32(2 ** 30)),
                     axis=1)
        min1_ref[0, pl.ds(i * bi, bi)] = jnp.maximum(m, 0.0)
        arg_ref[0, pl.ds(i * bi, bi)] = am

    @pl.when(i == ni - 1)
    def _():
        cm8 = cmin_ref[:, pl.ds(j * bj, bj)]
        cmin1 = jnp.maximum(jnp.min(cm8, axis=0), 0.0)
        pj = p_ref[0, pl.ds(j * bj, bj)]
        l1s_ref[0, 0] += jnp.sum(pj * cmin1)

    @pl.when(jnp.logical_and(i == ni - 1, j == nj - 1))
    def _():
        l1_ref[...] = jnp.reshape(l1s_ref[0, 0], (1, 1))


def _chamfer_main(xa, ya, probs, *, bi, bj, interpret=False):
    n = xa.shape[0]
    m = ya.shape[0]
    grid = (n // bi, m // bj)
    l1, min1, arg1 = pl.pallas_call(
        functools.partial(_main_kernel, bi=bi, bj=bj),
        grid=grid,
        in_specs=[
            pl.BlockSpec((bi, _KAUG), lambda i, j: (i, 0)),
            pl.BlockSpec((m, _KAUG), lambda i, j: (0, 0)),
            pl.BlockSpec((1, m), lambda i, j: (0, 0)),
        ],
        out_specs=[
            pl.BlockSpec((1, 1), lambda i, j: (0, 0)),
            pl.BlockSpec((1, n), lambda i, j: (0, 0)),
            pl.BlockSpec((1, n), lambda i, j: (0, 0)),
        ],
        out_shape=[
            jax.ShapeDtypeStruct((1, 1), jnp.float32),
            jax.ShapeDtypeStruct((1, n), jnp.float32),
            jax.ShapeDtypeStruct((1, n), jnp.int32),
        ],
        scratch_shapes=[
            pltpu.VMEM((8, m), jnp.float32),
            pltpu.VMEM((bi, 128), jnp.float32),
            pltpu.VMEM((bi, 128), jnp.int32),
            pltpu.SMEM((1, 1), jnp.float32),
        ],
        interpret=interpret,
    )(xa, ya, probs.reshape(1, m))
    return l1, min1, arg1


def _l2_sc(probs, min1, arg1):
    n = min1.shape[0]
    m = probs.shape[0]
    info = plsc.get_sparse_core_info()
    nc, ns = info.num_cores, info.num_subcores
    nw = nc * ns
    per = n // nw
    mesh = plsc.VectorSubcoreMesh(core_axis_name="c", subcore_axis_name="s")

    nchunk = per // 128

    @functools.partial(
        pl.kernel, mesh=mesh,
        out_type=jax.ShapeDtypeStruct((nw, 16), jnp.float32),
        scratch_types=[
            pltpu.VMEM((nchunk, 128), jnp.int32),
            pltpu.VMEM((per,), jnp.float32),
            pltpu.VMEM((per,), jnp.float32),
            pltpu.VMEM((16,), jnp.float32),
        ],
    )
    def k(probs_hbm, min1_hbm, arg_hbm, out_hbm, idx_v, g_v, m_v, acc_v):
        wid = jax.lax.axis_index("s") * nc + jax.lax.axis_index("c")
        base = wid * per
        pltpu.sync_copy(arg_hbm.at[pl.ds(wid * nchunk, nchunk)], idx_v)
        pltpu.sync_copy(min1_hbm.at[pl.ds(base, per)], m_v)
        for c in range(nchunk):
            pltpu.sync_copy(probs_hbm.at[idx_v.at[c]],
                            g_v.at[pl.ds(c * 128, 128)])

        def body(t, acc):
            g16 = g_v[pl.ds(t * 16, 16)]
            m16 = m_v[pl.ds(t * 16, 16)]
            return acc + g16 * m16

        acc_v[...] = jax.lax.fori_loop(0, per // 16, body,
                                       jnp.zeros((16,), jnp.float32))
        pltpu.sync_copy(acc_v, out_hbm.at[wid])

    return k(probs, min1, arg1.reshape(nw * nchunk, 128))


@jax.jit
def kernel(input_vertex_set, sampled_points, probs):
    n = input_vertex_set.shape[0]
    m = sampled_points.shape[0]
    xa, ya = _augment(input_vertex_set, sampled_points)
    bi = 1024 if n % 1024 == 0 else 256
    bj = 8192 if m % 8192 == 0 else 256
    l1, min1, arg1 = _chamfer_main(xa, ya, probs, bi=bi, bj=bj)
    l2p = _l2_sc(probs, min1[0], arg1[0])
    return l1[0, 0] + jnp.sum(l2p)


# speedup vs baseline: 1.0626x; 1.0152x over previous
import functools

import jax
import jax.numpy as jnp
from jax.experimental import pallas as pl
from jax.experimental.pallas import tpu as pltpu
from jax.experimental.pallas import tpu_sc as plsc

_NEG = -2.0
_BIG = 3.0e38
_KAUG = 136


def _augment_kernel(x_ref, y_ref, xa_ref, ya_ref, *, bp, d):
    x = x_ref[...]
    x2 = jnp.sum(x * x, axis=1, keepdims=True)
    ones = jnp.ones((bp, 1), jnp.float32)
    zeros = jnp.zeros((bp, _KAUG - d - 2), jnp.float32)
    xa_ref[...] = jnp.concatenate([_NEG * x, x2, ones, zeros], axis=1)
    y = y_ref[...]
    y2 = jnp.sum(y * y, axis=1, keepdims=True)
    ya_ref[...] = jnp.concatenate([y, ones, y2, zeros], axis=1)


def _augment(x, y, *, bp=512):
    n, d = x.shape
    out = pl.pallas_call(
        functools.partial(_augment_kernel, bp=bp, d=d),
        grid=(n // bp,),
        in_specs=[
            pl.BlockSpec((bp, d), lambda i: (i, 0)),
            pl.BlockSpec((bp, d), lambda i: (i, 0)),
        ],
        out_specs=[
            pl.BlockSpec((bp, _KAUG), lambda i: (i, 0)),
            pl.BlockSpec((bp, _KAUG), lambda i: (i, 0)),
        ],
        out_shape=[
            jax.ShapeDtypeStruct((n, _KAUG), jnp.float32),
            jax.ShapeDtypeStruct((n, _KAUG), jnp.float32),
        ],
    )(x, y)
    return out


def _main_kernel(xa_ref, ya_ref, p_ref, l1_ref, min1_ref, arg_ref,
                 cmin_ref, rmin_ref, ridx_ref, l1s_ref, *, bi, bj):
    i = pl.program_id(0)
    j = pl.program_id(1)
    ni = pl.num_programs(0)
    nj = pl.num_programs(1)
    ns = bj // 128

    @pl.when(jnp.logical_and(i == 0, j == 0))
    def _():
        cmin_ref[...] = jnp.full_like(cmin_ref, _BIG)
        l1s_ref[0, 0] = 0.0

    xa = xa_ref[...]

    @pl.when(j == 0)
    def _():
        rmin_ref[...] = jnp.full_like(rmin_ref, _BIG)
        ridx_ref[...] = jnp.zeros_like(ridx_ref)

    rm = rmin_ref[...]
    ri = ridx_ref[...]
    for s in range(ns):
        ya = ya_ref[pl.ds(j * bj + s * 128, 128), :]
        d_s = jax.lax.dot_general(
            xa, ya, (((1,), (1,)), ((), ())),
            preferred_element_type=jnp.float32)
        cpart = jnp.min(d_s.reshape(bi // 8, 8, 128), axis=0)
        cs = cmin_ref[:, pl.ds(j * bj + s * 128, 128)]
        cmin_ref[:, pl.ds(j * bj + s * 128, 128)] = jnp.minimum(cs, cpart)
        better = d_s < rm
        rm = jnp.where(better, d_s, rm)
        ri = jnp.where(better, jnp.int32(j * ns + s), ri)
    rmin_ref[...] = rm
    ridx_ref[...] = ri

    @pl.when(j == nj - 1)
    def _():
        m = jnp.min(rm, axis=1)
        lane = jax.lax.broadcasted_iota(jnp.int32, (bi, 128), 1)
        gidx = ri * 128 + lane
        am = jnp.min(jnp.where(rm == m[:, None], gidx, jnp.int32(2 ** 30)),
                     axis=1)
        min1_ref[0, pl.ds(i * bi, bi)] = jnp.maximum(m, 0.0)
        arg_ref[0, pl.ds(i * bi, bi)] = am

    @pl.when(i == ni - 1)
    def _():
        cm8 = cmin_ref[:, pl.ds(j * bj, bj)]
        cmin1 = jnp.maximum(jnp.min(cm8, axis=0), 0.0)
        pj = p_ref[0, pl.ds(j * bj, bj)]
        l1s_ref[0, 0] += jnp.sum(pj * cmin1)

    @pl.when(jnp.logical_and(i == ni - 1, j == nj - 1))
    def _():
        l1_ref[...] = jnp.reshape(l1s_ref[0, 0], (1, 1))


def _chamfer_main(xa, ya, probs, *, bi, bj, interpret=False):
    n = xa.shape[0]
    m = ya.shape[0]
    grid = (n // bi, m // bj)
    l1, min1, arg1 = pl.pallas_call(
        functools.partial(_main_kernel, bi=bi, bj=bj),
        grid=grid,
        in_specs=[
            pl.BlockSpec((bi, _KAUG), lambda i, j: (i, 0)),
            pl.BlockSpec((m, _KAUG), lambda i, j: (0, 0)),
            pl.BlockSpec((1, m), lambda i, j: (0, 0)),
        ],
        out_specs=[
            pl.BlockSpec((1, 1), lambda i, j: (0, 0)),
            pl.BlockSpec((1, n), lambda i, j: (0, 0)),
            pl.BlockSpec((1, n), lambda i, j: (0, 0)),
        ],
        out_shape=[
            jax.ShapeDtypeStruct((1, 1), jnp.float32),
            jax.ShapeDtypeStruct((1, n), jnp.float32),
            jax.ShapeDtypeStruct((1, n), jnp.int32),
        ],
        scratch_shapes=[
            pltpu.VMEM((8, m), jnp.float32),
            pltpu.VMEM((bi, 128), jnp.float32),
            pltpu.VMEM((bi, 128), jnp.int32),
            pltpu.SMEM((1, 1), jnp.float32),
        ],
        interpret=interpret,
    )(xa, ya, probs.reshape(1, m))
    return l1, min1, arg1


def _l2_sc(probs, min1, arg1):
    n = min1.shape[0]
    m = probs.shape[0]
    info = plsc.get_sparse_core_info()
    nc, ns = info.num_cores, info.num_subcores
    nw = nc * ns
    per = n // nw
    mesh = plsc.VectorSubcoreMesh(core_axis_name="c", subcore_axis_name="s")

    nchunk = per // 128

    @functools.partial(
        pl.kernel, mesh=mesh,
        out_type=jax.ShapeDtypeStruct((nw, 16), jnp.float32),
        scratch_types=[
            pltpu.VMEM((nchunk, 128), jnp.int32),
            pltpu.VMEM((per,), jnp.float32),
            pltpu.VMEM((per,), jnp.float32),
            pltpu.VMEM((16,), jnp.float32),
        ],
    )
    def k(probs_hbm, min1_hbm, arg_hbm, out_hbm, idx_v, g_v, m_v, acc_v):
        wid = jax.lax.axis_index("s") * nc + jax.lax.axis_index("c")
        base = wid * per
        pltpu.sync_copy(arg_hbm.at[pl.ds(wid * nchunk, nchunk)], idx_v)
        pltpu.sync_copy(min1_hbm.at[pl.ds(base, per)], m_v)
        for c in range(nchunk):
            pltpu.sync_copy(probs_hbm.at[idx_v.at[c]],
                            g_v.at[pl.ds(c * 128, 128)])

        def body(t, acc):
            g16 = g_v[pl.ds(t * 16, 16)]
            m16 = m_v[pl.ds(t * 16, 16)]
            return acc + g16 * m16

        acc_v[...] = jax.lax.fori_loop(0, per // 16, body,
                                       jnp.zeros((16,), jnp.float32))
        pltpu.sync_copy(acc_v, out_hbm.at[wid])

    return k(probs, min1, arg1.reshape(nw * nchunk, 128))


@jax.jit
def kernel(input_vertex_set, sampled_points, probs):
    n = input_vertex_set.shape[0]
    m = sampled_points.shape[0]
    xa, ya = _augment(input_vertex_set, sampled_points)
    bi = 2048 if n % 2048 == 0 else 256
    bj = 8192 if m % 8192 == 0 else 256
    l1, min1, arg1 = _chamfer_main(xa, ya, probs, bi=bi, bj=bj)
    l2p = _l2_sc(probs, min1[0], arg1[0])
    return l1[0, 0] + jnp.sum(l2p)


# speedup vs baseline: 1.0850x; 1.0211x over previous
import functools

import jax
import jax.numpy as jnp
from jax.experimental import pallas as pl
from jax.experimental.pallas import tpu as pltpu
from jax.experimental.pallas import tpu_sc as plsc

_NEG = -2.0
_BIG = 3.0e38
_KAUG = 136


def _augment_kernel(x_ref, y_ref, xa_ref, ya_ref, *, bp, d):
    ones = jnp.ones((bp, 1), jnp.bfloat16)
    zeros = jnp.zeros((bp, _KAUG - d - 4), jnp.bfloat16)
    x = x_ref[...]
    x2 = jnp.sum(x * x, axis=1, keepdims=True)
    x2hi = x2.astype(jnp.bfloat16)
    x2lo = (x2 - x2hi.astype(jnp.float32)).astype(jnp.bfloat16)
    xa_ref[...] = jnp.concatenate(
        [(_NEG * x).astype(jnp.bfloat16), x2hi, x2lo, ones, ones, zeros],
        axis=1)
    y = y_ref[...]
    y2 = jnp.sum(y * y, axis=1, keepdims=True)
    y2hi = y2.astype(jnp.bfloat16)
    y2lo = (y2 - y2hi.astype(jnp.float32)).astype(jnp.bfloat16)
    ya_ref[...] = jnp.concatenate(
        [y.astype(jnp.bfloat16), ones, ones, y2hi, y2lo, zeros], axis=1)


def _augment(x, y, *, bp=512):
    n, d = x.shape
    out = pl.pallas_call(
        functools.partial(_augment_kernel, bp=bp, d=d),
        grid=(n // bp,),
        in_specs=[
            pl.BlockSpec((bp, d), lambda i: (i, 0)),
            pl.BlockSpec((bp, d), lambda i: (i, 0)),
        ],
        out_specs=[
            pl.BlockSpec((bp, _KAUG), lambda i: (i, 0)),
            pl.BlockSpec((bp, _KAUG), lambda i: (i, 0)),
        ],
        out_shape=[
            jax.ShapeDtypeStruct((n, _KAUG), jnp.bfloat16),
            jax.ShapeDtypeStruct((n, _KAUG), jnp.bfloat16),
        ],
    )(x, y)
    return out


def _main_kernel(xa_ref, ya_ref, p_ref, l1_ref, min1_ref, arg_ref,
                 cmin_ref, rmin_ref, ridx_ref, l1s_ref, *, bi, bj):
    i = pl.program_id(0)
    j = pl.program_id(1)
    ni = pl.num_programs(0)
    nj = pl.num_programs(1)
    ns = bj // 128

    @pl.when(jnp.logical_and(i == 0, j == 0))
    def _():
        cmin_ref[...] = jnp.full_like(cmin_ref, _BIG)
        l1s_ref[0, 0] = 0.0

    xa = xa_ref[...]

    @pl.when(j == 0)
    def _():
        rmin_ref[...] = jnp.full_like(rmin_ref, _BIG)
        ridx_ref[...] = jnp.zeros_like(ridx_ref)

    rm = rmin_ref[...]
    ri = ridx_ref[...]
    for s in range(ns):
        ya = ya_ref[pl.ds(j * bj + s * 128, 128), :]
        d_s = jax.lax.dot_general(
            xa, ya, (((1,), (1,)), ((), ())),
            preferred_element_type=jnp.float32)
        cpart = jnp.min(d_s.reshape(bi // 8, 8, 128), axis=0)
        cs = cmin_ref[:, pl.ds(j * bj + s * 128, 128)]
        cmin_ref[:, pl.ds(j * bj + s * 128, 128)] = jnp.minimum(cs, cpart)
        better = d_s < rm
        rm = jnp.where(better, d_s, rm)
        ri = jnp.where(better, jnp.int32(j * ns + s), ri)
    rmin_ref[...] = rm
    ridx_ref[...] = ri

    @pl.when(j == nj - 1)
    def _():
        m = jnp.min(rm, axis=1)
        lane = jax.lax.broadcasted_iota(jnp.int32, (bi, 128), 1)
        gidx = ri * 128 + lane
        am = jnp.min(jnp.where(rm == m[:, None], gidx, jnp.int32(2 ** 30)),
                     axis=1)
        min1_ref[0, pl.ds(i * bi, bi)] = jnp.maximum(m, 0.0)
        arg_ref[0, pl.ds(i * bi, bi)] = am

    @pl.when(i == ni - 1)
    def _():
        cm8 = cmin_ref[:, pl.ds(j * bj, bj)]
        cmin1 = jnp.maximum(jnp.min(cm8, axis=0), 0.0)
        pj = p_ref[0, pl.ds(j * bj, bj)]
        l1s_ref[0, 0] += jnp.sum(pj * cmin1)

    @pl.when(jnp.logical_and(i == ni - 1, j == nj - 1))
    def _():
        l1_ref[...] = jnp.reshape(l1s_ref[0, 0], (1, 1))


def _chamfer_main(xa, ya, probs, *, bi, bj, interpret=False):
    n = xa.shape[0]
    m = ya.shape[0]
    grid = (n // bi, m // bj)
    l1, min1, arg1 = pl.pallas_call(
        functools.partial(_main_kernel, bi=bi, bj=bj),
        grid=grid,
        in_specs=[
            pl.BlockSpec((bi, _KAUG), lambda i, j: (i, 0)),
            pl.BlockSpec((m, _KAUG), lambda i, j: (0, 0)),
            pl.BlockSpec((1, m), lambda i, j: (0, 0)),
        ],
        out_specs=[
            pl.BlockSpec((1, 1), lambda i, j: (0, 0)),
            pl.BlockSpec((1, n), lambda i, j: (0, 0)),
            pl.BlockSpec((1, n), lambda i, j: (0, 0)),
        ],
        out_shape=[
            jax.ShapeDtypeStruct((1, 1), jnp.float32),
            jax.ShapeDtypeStruct((1, n), jnp.float32),
            jax.ShapeDtypeStruct((1, n), jnp.int32),
        ],
        scratch_shapes=[
            pltpu.VMEM((8, m), jnp.float32),
            pltpu.VMEM((bi, 128), jnp.float32),
            pltpu.VMEM((bi, 128), jnp.int32),
            pltpu.SMEM((1, 1), jnp.float32),
        ],
        interpret=interpret,
    )(xa, ya, probs.reshape(1, m))
    return l1, min1, arg1


def _l2_sc(probs, min1, arg1):
    n = min1.shape[0]
    m = probs.shape[0]
    info = plsc.get_sparse_core_info()
    nc, ns = info.num_cores, info.num_subcores
    nw = nc * ns
    per = n // nw
    mesh = plsc.VectorSubcoreMesh(core_axis_name="c", subcore_axis_name="s")

    nchunk = per // 128

    @functools.partial(
        pl.kernel, mesh=mesh,
        out_type=jax.ShapeDtypeStruct((nw, 16), jnp.float32),
        scratch_types=[
            pltpu.VMEM((nchunk, 128), jnp.int32),
            pltpu.VMEM((per,), jnp.float32),
            pltpu.VMEM((per,), jnp.float32),
            pltpu.VMEM((16,), jnp.float32),
        ],
    )
    def k(probs_hbm, min1_hbm, arg_hbm, out_hbm, idx_v, g_v, m_v, acc_v):
        wid = jax.lax.axis_index("s") * nc + jax.lax.axis_index("c")
        base = wid * per
        pltpu.sync_copy(arg_hbm.at[pl.ds(wid * nchunk, nchunk)], idx_v)
        pltpu.sync_copy(min1_hbm.at[pl.ds(base, per)], m_v)
        for c in range(nchunk):
            pltpu.sync_copy(probs_hbm.at[idx_v.at[c]],
                            g_v.at[pl.ds(c * 128, 128)])

        def body(t, acc):
            g16 = g_v[pl.ds(t * 16, 16)]
            m16 = m_v[pl.ds(t * 16, 16)]
            return acc + g16 * m16

        acc_v[...] = jax.lax.fori_loop(0, per // 16, body,
                                       jnp.zeros((16,), jnp.float32))
        pltpu.sync_copy(acc_v, out_hbm.at[wid])

    return k(probs, min1, arg1.reshape(nw * nchunk, 128))


@jax.jit
def kernel(input_vertex_set, sampled_points, probs):
    n = input_vertex_set.shape[0]
    m = sampled_points.shape[0]
    xa, ya = _augment(input_vertex_set, sampled_points)
    bi = 2048 if n % 2048 == 0 else 256
    bj = 8192 if m % 8192 == 0 else 256
    l1, min1, arg1 = _chamfer_main(xa, ya, probs, bi=bi, bj=bj)
    l2p = _l2_sc(probs, min1[0], arg1[0])
    return l1[0, 0] + jnp.sum(l2p)


# speedup vs baseline: 1.7076x; 1.5738x over previous
import functools

import jax
import jax.numpy as jnp
from jax.experimental import pallas as pl
from jax.experimental.pallas import tpu as pltpu
from jax.experimental.pallas import tpu_sc as plsc

_NEG = -2.0
_BIG = 3.0e38
_KAUG = 136


def _augment_kernel(x_ref, y_ref, xa_ref, ya_ref, *, bp, d):
    ones = jnp.ones((bp, 1), jnp.bfloat16)
    zeros = jnp.zeros((bp, _KAUG - d - 4), jnp.bfloat16)
    x = x_ref[...]
    x2 = jnp.sum(x * x, axis=1, keepdims=True)
    x2hi = x2.astype(jnp.bfloat16)
    x2lo = (x2 - x2hi.astype(jnp.float32)).astype(jnp.bfloat16)
    xa_ref[...] = jnp.concatenate(
        [(_NEG * x).astype(jnp.bfloat16), x2hi, x2lo, ones, ones, zeros],
        axis=1)
    y = y_ref[...]
    y2 = jnp.sum(y * y, axis=1, keepdims=True)
    y2hi = y2.astype(jnp.bfloat16)
    y2lo = (y2 - y2hi.astype(jnp.float32)).astype(jnp.bfloat16)
    ya_ref[...] = jnp.concatenate(
        [y.astype(jnp.bfloat16), ones, ones, y2hi, y2lo, zeros], axis=1)


def _augment(x, y, *, bp=512):
    n, d = x.shape
    out = pl.pallas_call(
        functools.partial(_augment_kernel, bp=bp, d=d),
        grid=(n // bp,),
        in_specs=[
            pl.BlockSpec((bp, d), lambda i: (i, 0)),
            pl.BlockSpec((bp, d), lambda i: (i, 0)),
        ],
        out_specs=[
            pl.BlockSpec((bp, _KAUG), lambda i: (i, 0)),
            pl.BlockSpec((bp, _KAUG), lambda i: (i, 0)),
        ],
        out_shape=[
            jax.ShapeDtypeStruct((n, _KAUG), jnp.bfloat16),
            jax.ShapeDtypeStruct((n, _KAUG), jnp.bfloat16),
        ],
    )(x, y)
    return out


def _main_kernel(xa_ref, ya_ref, p_ref, l1_ref, min1_ref, arg_ref,
                 cmin_ref, rmin_ref, ridx_ref, l1s_ref, *, bi, bj):
    i = pl.program_id(0)
    j = pl.program_id(1)
    ni = pl.num_programs(0)
    nj = pl.num_programs(1)
    ns = bj // 128

    @pl.when(jnp.logical_and(i == 0, j == 0))
    def _():
        cmin_ref[...] = jnp.full_like(cmin_ref, _BIG)
        l1s_ref[0, 0] = 0.0

    xa = xa_ref[...]

    @pl.when(j == 0)
    def _():
        rmin_ref[...] = jnp.full_like(rmin_ref, _BIG)
        ridx_ref[...] = jnp.zeros_like(ridx_ref)

    rm = rmin_ref[...]
    ri = ridx_ref[...]
    bw = 512
    for w in range(bj // bw):
        ya = ya_ref[pl.ds(j * bj + w * bw, bw), :]
        d_w = jax.lax.dot_general(
            xa, ya, (((1,), (1,)), ((), ())),
            preferred_element_type=jnp.float32)
        for t in range(bw // 128):
            s = w * (bw // 128) + t
            d_s = d_w[:, t * 128:(t + 1) * 128]
            cpart = jnp.min(d_s.reshape(bi // 8, 8, 128), axis=0)
            cs = cmin_ref[:, pl.ds(j * bj + s * 128, 128)]
            cmin_ref[:, pl.ds(j * bj + s * 128, 128)] = jnp.minimum(cs, cpart)
            better = d_s < rm
            rm = jnp.where(better, d_s, rm)
            ri = jnp.where(better, jnp.int32(j * ns + s), ri)
    rmin_ref[...] = rm
    ridx_ref[...] = ri

    @pl.when(j == nj - 1)
    def _():
        m = jnp.min(rm, axis=1)
        lane = jax.lax.broadcasted_iota(jnp.int32, (bi, 128), 1)
        gidx = ri * 128 + lane
        am = jnp.min(jnp.where(rm == m[:, None], gidx, jnp.int32(2 ** 30)),
                     axis=1)
        min1_ref[0, pl.ds(i * bi, bi)] = jnp.maximum(m, 0.0)
        arg_ref[0, pl.ds(i * bi, bi)] = am

    @pl.when(i == ni - 1)
    def _():
        cm8 = cmin_ref[:, pl.ds(j * bj, bj)]
        cmin1 = jnp.maximum(jnp.min(cm8, axis=0), 0.0)
        pj = p_ref[0, pl.ds(j * bj, bj)]
        l1s_ref[0, 0] += jnp.sum(pj * cmin1)

    @pl.when(jnp.logical_and(i == ni - 1, j == nj - 1))
    def _():
        l1_ref[...] = jnp.reshape(l1s_ref[0, 0], (1, 1))


def _chamfer_main(xa, ya, probs, *, bi, bj, interpret=False):
    n = xa.shape[0]
    m = ya.shape[0]
    grid = (n // bi, m // bj)
    l1, min1, arg1 = pl.pallas_call(
        functools.partial(_main_kernel, bi=bi, bj=bj),
        grid=grid,
        in_specs=[
            pl.BlockSpec((bi, _KAUG), lambda i, j: (i, 0)),
            pl.BlockSpec((m, _KAUG), lambda i, j: (0, 0)),
            pl.BlockSpec((1, m), lambda i, j: (0, 0)),
        ],
        out_specs=[
            pl.BlockSpec((1, 1), lambda i, j: (0, 0)),
            pl.BlockSpec((1, n), lambda i, j: (0, 0)),
            pl.BlockSpec((1, n), lambda i, j: (0, 0)),
        ],
        out_shape=[
            jax.ShapeDtypeStruct((1, 1), jnp.float32),
            jax.ShapeDtypeStruct((1, n), jnp.float32),
            jax.ShapeDtypeStruct((1, n), jnp.int32),
        ],
        scratch_shapes=[
            pltpu.VMEM((8, m), jnp.float32),
            pltpu.VMEM((bi, 128), jnp.float32),
            pltpu.VMEM((bi, 128), jnp.int32),
            pltpu.SMEM((1, 1), jnp.float32),
        ],
        interpret=interpret,
    )(xa, ya, probs.reshape(1, m))
    return l1, min1, arg1


def _l2_sc(probs, min1, arg1):
    n = min1.shape[0]
    m = probs.shape[0]
    info = plsc.get_sparse_core_info()
    nc, ns = info.num_cores, info.num_subcores
    nw = nc * ns
    per = n // nw
    mesh = plsc.VectorSubcoreMesh(core_axis_name="c", subcore_axis_name="s")

    nchunk = per // 128

    @functools.partial(
        pl.kernel, mesh=mesh,
        out_type=jax.ShapeDtypeStruct((nw, 16), jnp.float32),
        scratch_types=[
            pltpu.VMEM((nchunk, 128), jnp.int32),
            pltpu.VMEM((per,), jnp.float32),
            pltpu.VMEM((per,), jnp.float32),
            pltpu.VMEM((16,), jnp.float32),
        ],
    )
    def k(probs_hbm, min1_hbm, arg_hbm, out_hbm, idx_v, g_v, m_v, acc_v):
        wid = jax.lax.axis_index("s") * nc + jax.lax.axis_index("c")
        base = wid * per
        pltpu.sync_copy(arg_hbm.at[pl.ds(wid * nchunk, nchunk)], idx_v)
        pltpu.sync_copy(min1_hbm.at[pl.ds(base, per)], m_v)
        for c in range(nchunk):
            pltpu.sync_copy(probs_hbm.at[idx_v.at[c]],
                            g_v.at[pl.ds(c * 128, 128)])

        def body(t, acc):
            g16 = g_v[pl.ds(t * 16, 16)]
            m16 = m_v[pl.ds(t * 16, 16)]
            return acc + g16 * m16

        acc_v[...] = jax.lax.fori_loop(0, per // 16, body,
                                       jnp.zeros((16,), jnp.float32))
        pltpu.sync_copy(acc_v, out_hbm.at[wid])

    return k(probs, min1, arg1.reshape(nw * nchunk, 128))


@jax.jit
def kernel(input_vertex_set, sampled_points, probs):
    n = input_vertex_set.shape[0]
    m = sampled_points.shape[0]
    xa, ya = _augment(input_vertex_set, sampled_points)
    bi = 2048 if n % 2048 == 0 else 256
    bj = 8192 if m % 8192 == 0 else 256
    l1, min1, arg1 = _chamfer_main(xa, ya, probs, bi=bi, bj=bj)
    l2p = _l2_sc(probs, min1[0], arg1[0])
    return l1[0, 0] + jnp.sum(l2p)
